# Initial kernel scaffold; baseline (speedup 1.0000x reference)
import functools

import jax
import jax.numpy as jnp
from jax import lax
from jax.experimental import pallas as pl
from jax.experimental.pallas import tpu as pltpu
from jax.experimental.pallas import tpu_sc as plsc

N = 10000
E = 320000
R = 1000
D = 128
NC = 2
NS = 16
EPT = E // NS
B = 80
NBLK = EPT // B
G = B // 16
RPT = N // NS
QPAD = 1024
FBN = 2000


def _mv_body(xe_ref, xr_ref, wh_ref, wt_ref, wr_ref, p2_ref, q_ref):
    xe = xe_ref[...]
    p2_ref[0, :] = jnp.sum(xe * wh_ref[...][None, :], axis=1)
    p2_ref[1, :] = jnp.sum(xe * wt_ref[...][None, :], axis=1)
    qq = jnp.sum(xr_ref[...] * wr_ref[...][None, :], axis=1)
    q_ref[0, :] = jnp.concatenate([qq, jnp.zeros((QPAD - R,), jnp.float32)])


_mv_call = pl.pallas_call(
    _mv_body,
    out_shape=[
        jax.ShapeDtypeStruct((NC, N), jnp.float32),
        jax.ShapeDtypeStruct((1, QPAD), jnp.float32),
    ],
)


def _sc_body(p2, qpad, dst2, rel, xr, acc_out, s_out,
             acc_sh, ssh, ptab, qtab, dloc, rloc, exb, svec, rows,
             zbuf, sbuf, sem):
    c = lax.axis_index("c")
    t = lax.axis_index("s")

    pltpu.sync_copy(p2.at[c], ptab)
    pltpu.sync_copy(qpad, qtab)

    zv = jnp.zeros((16,), jnp.float32)

    def _zrow(i, carry):
        for k in range(D // 16):
            zbuf[i, pl.ds(k * 16, 16)] = zv
        sbuf[i, :] = zv
        return carry

    lax.fori_loop(0, RPT, _zrow, 0)

    def _zsvec(i, carry):
        svec[i, :] = zv
        return carry

    lax.fori_loop(0, B, _zsvec, 0)

    base_r = t * RPT
    pltpu.sync_copy(zbuf, acc_sh.at[pl.ds(base_r, RPT)])
    pltpu.sync_copy(sbuf, ssh.at[pl.ds(base_r, RPT)])
    plsc.subcore_barrier()

    ebase = t * EPT
    lane0 = jnp.zeros((16,), jnp.int32)

    def _blk(b, carry):
        off = ebase + b * B
        pltpu.sync_copy(dst2.at[c, pl.ds(off, B)], dloc)
        pltpu.sync_copy(rel.at[pl.ds(off, B)], rloc)
        cp = pltpu.async_copy(xr.at[rloc], rows, sem)
        for g in range(G):
            di = dloc[pl.ds(g * 16, 16)]
            ri = rloc[pl.ds(g * 16, 16)]
            z = plsc.load_gather(ptab, [di]) + plsc.load_gather(qtab, [ri])
            z = jnp.maximum(z, z * 0.01)
            ex = jnp.exp(z)
            exb[pl.ds(g * 16, 16)] = ex
            rowi = lax.iota(jnp.int32, 16) + (g * 16)
            plsc.store_scatter(svec, [rowi, lane0], ex)
        cp.wait()
        for j in range(B):
            sc = plsc.load_gather(exb, [jnp.full((16,), j, jnp.int32)])
            for k in range(D // 16):
                rows[j, pl.ds(k * 16, 16)] = rows[j, pl.ds(k * 16, 16)] * sc
        pltpu.sync_copy(rows, acc_sh.at[dloc], add=True)
        pltpu.sync_copy(svec, ssh.at[dloc], add=True)
        return carry

    lax.fori_loop(0, NBLK, _blk, 0)

    plsc.subcore_barrier()
    pltpu.sync_copy(acc_sh.at[pl.ds(base_r, RPT)], zbuf)
    pltpu.sync_copy(zbuf, acc_out.at[c, pl.ds(base_r, RPT)])
    pltpu.sync_copy(ssh.at[pl.ds(base_r, RPT)], sbuf)
    pltpu.sync_copy(sbuf, s_out.at[c, pl.ds(base_r, RPT)])


_sc_call = pl.kernel(
    _sc_body,
    out_type=[
        jax.ShapeDtypeStruct((NC, N, D), jnp.float32),
        jax.ShapeDtypeStruct((NC, N, 16), jnp.float32),
    ],
    mesh=plsc.VectorSubcoreMesh(
        core_axis_name="c", subcore_axis_name="s",
        num_cores=NC, num_subcores=NS,
    ),
    scratch_types=[
        pltpu.VMEM_SHARED((N, D), jnp.float32),
        pltpu.VMEM_SHARED((N, 16), jnp.float32),
        pltpu.VMEM((N,), jnp.float32),
        pltpu.VMEM((QPAD,), jnp.float32),
        pltpu.VMEM((B,), jnp.int32),
        pltpu.VMEM((B,), jnp.int32),
        pltpu.VMEM((B,), jnp.float32),
        pltpu.VMEM((B, 16), jnp.float32),
        pltpu.VMEM((B, D), jnp.float32),
        pltpu.VMEM((RPT, D), jnp.float32),
        pltpu.VMEM((RPT, 16), jnp.float32),
        pltpu.SemaphoreType.DMA,
    ],
)


def _fin_body(acc_ref, ssh_ref, out_ref):
    s = jnp.sum(ssh_ref[...], axis=2)
    inv_h = 1.0 / (s[0] + 1e-16)
    inv_t = 1.0 / (s[1] + 1e-16)
    out_ref[...] = jnp.concatenate(
        [acc_ref[0] * inv_h[:, None], acc_ref[1] * inv_t[:, None]], axis=1)


_fin_call = pl.pallas_call(
    _fin_body,
    grid=(N // FBN,),
    in_specs=[
        pl.BlockSpec((NC, FBN, D), lambda i: (0, i, 0)),
        pl.BlockSpec((NC, FBN, 16), lambda i: (0, i, 0)),
    ],
    out_specs=pl.BlockSpec((FBN, 2 * D), lambda i: (i, 0)),
    out_shape=jax.ShapeDtypeStruct((N, 2 * D), jnp.float32),
)


def kernel(x_e, x_r, edge_index, rel, W_h, W_t, W_r):
    p2, q2 = _mv_call(x_e, x_r, W_h, W_t, W_r)
    qpad = q2.reshape(QPAD)
    acc, ssum = _sc_call(p2, qpad, edge_index, rel, x_r)
    return _fin_call(acc, ssum)


# speedup vs baseline: 21.5656x; 21.5656x over previous
import functools

import jax
import jax.numpy as jnp
from jax import lax
from jax.experimental import pallas as pl
from jax.experimental.pallas import tpu as pltpu
from jax.experimental.pallas import tpu_sc as plsc

N = 10000
E = 320000
R = 1000
D = 128
NC = 2
NS = 16
NPAD = 10240
EPT = E // NS
B = 80
NBLK = EPT // B
G = B // 16
RPT = NPAD // NS
DCH = B
SCH = B
QPAD = 1024
FBN = 2000


def _mv_body(xe_ref, xr_ref, wh_ref, wt_ref, wr_ref, p2_ref, q_ref):
    xe = xe_ref[...]
    p2_ref[0, :] = jnp.sum(xe * wh_ref[...][None, :], axis=1)
    p2_ref[1, :] = jnp.sum(xe * wt_ref[...][None, :], axis=1)
    qq = jnp.sum(xr_ref[...] * wr_ref[...][None, :], axis=1)
    q_ref[0, :] = jnp.concatenate([qq, jnp.zeros((QPAD - R,), jnp.float32)])


_mv_call = pl.pallas_call(
    _mv_body,
    out_shape=[
        jax.ShapeDtypeStruct((NC, N), jnp.float32),
        jax.ShapeDtypeStruct((1, QPAD), jnp.float32),
    ],
)


def _sc_body(pflat, qpad, dstflat, rel, xr, acc_out, s_out,
             acc_sh, ptab, qtab, spart, dloc, rloc, exb, rows, sem):
    c = lax.axis_index("c")
    t = lax.axis_index("s")

    pltpu.sync_copy(pflat.at[pl.ds(c * N, N)], ptab)
    pltpu.sync_copy(qpad, qtab)

    zv = jnp.zeros((16,), jnp.float32)

    def _zrow(i, carry):
        for k in range(D // 16):
            rows[i, pl.ds(k * 16, 16)] = zv
        return carry

    lax.fori_loop(0, B, _zrow, 0)

    def _zs(i, carry):
        spart[pl.ds(i * 16, 16)] = zv
        return carry

    lax.fori_loop(0, N // 16, _zs, 0)

    base_r = t * RPT

    def _set_idx(base):
        for g in range(G):
            dloc[pl.ds(g * 16, 16)] = lax.iota(jnp.int32, 16) + (base + g * 16)

    for k in range(RPT // DCH):
        _set_idx(base_r + k * DCH)
        pltpu.sync_copy(rows, acc_sh.at[dloc])
    plsc.subcore_barrier()

    ebase = t * EPT
    lanes = lax.iota(jnp.int32, 16)

    def _blk(b, carry):
        off = b * B
        pltpu.sync_copy(dstflat.at[pl.ds(c * E + ebase + off, B)], dloc)
        pltpu.sync_copy(rel.at[pl.ds(ebase + off, B)], rloc)
        cp = pltpu.async_copy(xr.at[rloc], rows, sem)
        for g in range(G):
            di = dloc[pl.ds(g * 16, 16)]
            ri = rloc[pl.ds(g * 16, 16)]
            z = plsc.load_gather(ptab, [di]) + plsc.load_gather(qtab, [ri])
            z = jnp.maximum(z, z * 0.01)
            ex = jnp.exp(z)
            exb[pl.ds(16 + g * 16, 16)] = ex
            for l in range(16):
                plsc.addupdate_scatter(spart, [di], ex, mask=lanes == l)
        cp.wait()
        for j in range(B):
            sc = plsc.load_gather(exb, [jnp.full((16,), 16 + j, jnp.int32)])
            for k in range(D // 16):
                rows[j, pl.ds(k * 16, 16)] = rows[j, pl.ds(k * 16, 16)] * sc
        pltpu.sync_copy(rows, acc_sh.at[dloc], add=True)
        return carry

    lax.fori_loop(0, NBLK, _blk, 0)

    plsc.subcore_barrier()
    for k in range(RPT // DCH):
        _set_idx(base_r + k * DCH)
        pltpu.sync_copy(acc_sh.at[dloc], rows)
        pltpu.sync_copy(rows, acc_out.at[c, pl.ds(base_r + k * DCH, DCH)])
    pltpu.sync_copy(spart, s_out.at[pl.ds((c * NS + t) * N, N)])


_sc_call = pl.kernel(
    _sc_body,
    out_type=[
        jax.ShapeDtypeStruct((NC, NPAD, D), jnp.float32),
        jax.ShapeDtypeStruct((NC * NS * N,), jnp.float32),
    ],
    mesh=plsc.VectorSubcoreMesh(
        core_axis_name="c", subcore_axis_name="s",
        num_cores=NC, num_subcores=NS,
    ),
    compiler_params=pltpu.CompilerParams(needs_layout_passes=False),
    scratch_types=[
        pltpu.VMEM_SHARED((NPAD, D), jnp.float32),
        pltpu.VMEM((N,), jnp.float32),
        pltpu.VMEM((QPAD,), jnp.float32),
        pltpu.VMEM((N,), jnp.float32),
        pltpu.VMEM((B,), jnp.int32),
        pltpu.VMEM((B,), jnp.int32),
        pltpu.VMEM((B + 16,), jnp.float32),
        pltpu.VMEM((B, D), jnp.float32),
        pltpu.SemaphoreType.DMA,
    ],
)


def _fin_body(acc_ref, ssh_ref, out_ref):
    s = jnp.sum(ssh_ref[...], axis=1)
    inv_h = 1.0 / (s[0] + 1e-16)
    inv_t = 1.0 / (s[1] + 1e-16)
    out_ref[...] = jnp.concatenate(
        [acc_ref[0, :N] * inv_h[:, None], acc_ref[1, :N] * inv_t[:, None]],
        axis=1)


_fin_call = pl.pallas_call(
    _fin_body,
    out_shape=jax.ShapeDtypeStruct((N, 2 * D), jnp.float32),
)


def kernel(x_e, x_r, edge_index, rel, W_h, W_t, W_r):
    p2, q2 = _mv_call(x_e, x_r, W_h, W_t, W_r)
    qpad = q2.reshape(QPAD)
    acc, ssum = _sc_call(p2.reshape(NC * N), qpad, edge_index.reshape(NC * E),
                         rel, x_r)
    return _fin_call(acc, ssum.reshape(NC, NS, N))
